# Initial kernel scaffold; baseline (speedup 1.0000x reference)
import jax
import jax.numpy as jnp
from jax.experimental import pallas as pl


def kernel(x, x_0, edge_index, W, b, gamma, beta, running_mean, running_var):
    raise NotImplementedError("write your pallas kernel here")


# speedup vs baseline: 928.9229x; 928.9229x over previous
import functools

import jax
import jax.numpy as jnp
from jax import lax
from jax.experimental import pallas as pl
from jax.experimental.pallas import tpu as pltpu
from jax.experimental.pallas import tpu_sc as plsc

_NC = 2
_NS = 16
_NW = _NC * _NS
_LANES = 16

_CHUNK = 8


def _mm_body(x_ref, a_ref, v_ref, u_out, v_out):
    xb = x_ref[...]
    u_out[...] = lax.dot_general(xb, a_ref[...], (((0,), (1,)), ((), ())),
                                 preferred_element_type=jnp.float32)
    v_out[...] = lax.dot_general(xb, v_ref[...], (((0,), (1,)), ((), ())),
                                 preferred_element_type=jnp.float32)


def _node_tables(xp, a, vw, np_, c, out_c, bn=512):
    grid = (np_ // bn,)
    return pl.pallas_call(
        _mm_body,
        grid=grid,
        in_specs=[
            pl.BlockSpec((c, bn), lambda i: (0, i)),
            pl.BlockSpec((out_c, c), lambda i: (0, 0)),
            pl.BlockSpec((out_c, c), lambda i: (0, 0)),
        ],
        out_specs=[
            pl.BlockSpec((bn, out_c), lambda i: (i, 0)),
            pl.BlockSpec((bn, out_c), lambda i: (i, 0)),
        ],
        out_shape=[
            jax.ShapeDtypeStruct((np_, out_c), jnp.float32),
            jax.ShapeDtypeStruct((np_, out_c), jnp.float32),
        ],
    )(xp, a, vw)


def _sc_combine(u_t, v_t, iu, iv, shift, np_, ke, out_c):
    nodes_per_w = np_ // _NW
    nchunks = nodes_per_w // _CHUNK
    nidx = _CHUNK * ke
    ngrp = out_c // _LANES

    mesh = plsc.VectorSubcoreMesh(core_axis_name="c", subcore_axis_name="s")

    @functools.partial(
        pl.kernel,
        out_type=jax.ShapeDtypeStruct((np_, out_c), jnp.float32),
        mesh=mesh,
        scratch_types=[
            pltpu.VMEM((nidx,), jnp.int32),
            pltpu.VMEM((nidx,), jnp.int32),
            pltpu.VMEM((nidx, out_c), jnp.float32),
            pltpu.VMEM((nidx, out_c), jnp.float32),
            pltpu.VMEM((_CHUNK, out_c), jnp.float32),
            pltpu.VMEM((out_c,), jnp.float32),
            pltpu.SemaphoreType.DMA,
            pltpu.SemaphoreType.DMA,
        ],
    )
    def sc_kernel(u_hbm, v_hbm, iu_hbm, iv_hbm, shift_hbm, out_hbm,
                  iu_v, iv_v, urows, vrows, outrows, shift_v, sem_u, sem_v):
        wid = lax.axis_index("s") * _NC + lax.axis_index("c")
        base = wid * nodes_per_w
        pltpu.sync_copy(shift_hbm, shift_v)

        def chunk_body(ci, carry):
            nb = base + ci * _CHUNK
            pltpu.sync_copy(iu_hbm.at[pl.ds(nb * ke, nidx)], iu_v)
            pltpu.sync_copy(iv_hbm.at[pl.ds(nb * ke, nidx)], iv_v)
            cu = pltpu.async_copy(u_hbm.at[iu_v], urows, sem_u)
            cv = pltpu.async_copy(v_hbm.at[iv_v], vrows, sem_v)
            cu.wait()
            cv.wait()

            def node_body(n, carry2):
                e0 = n * ke
                for g in range(ngrp):
                    sl = pl.ds(g * _LANES, _LANES)
                    m = urows[e0, sl] + vrows[e0, sl]
                    for e in range(1, ke):
                        m = jnp.maximum(m, urows[e0 + e, sl] + vrows[e0 + e, sl])
                    outrows[n, sl] = jnp.maximum(m + shift_v[sl], 0.0)
                return carry2

            lax.fori_loop(0, _CHUNK, node_body, 0, unroll=False)
            pltpu.sync_copy(outrows, out_hbm.at[pl.ds(nb, _CHUNK)])
            return carry

        lax.fori_loop(0, nchunks, chunk_body, 0, unroll=False)

    return sc_kernel(u_t, v_t, iu, iv, shift)


def kernel(x, x_0, edge_index, W, b, gamma, beta, running_mean, running_var):
    _, c, n, _ = x.shape
    out_c = W.shape[0]
    k = edge_index.shape[-1]
    ke = k - 1

    scale = gamma / jnp.sqrt(running_var + 1e-5)
    shift = (b - running_mean) * scale + beta
    w1 = W[:, :c]
    w2 = W[:, c:]
    a = (w1 - w2) * scale[:, None]
    vw = w2 * scale[:, None]

    align = _NW * _CHUNK
    np_ = ((n + align - 1) // align) * align

    xt = x[0, :, :, 0]
    xp = jnp.pad(xt, ((0, 0), (0, np_ - n)))

    u_t, v_t = _node_tables(xp, a, vw, np_, c, out_c)

    ei = edge_index.astype(jnp.int32)
    iu = jnp.pad(ei[1, 0, :, 1:], ((0, np_ - n), (0, 0))).reshape(-1)
    iv = jnp.pad(ei[0, 0, :, 1:], ((0, np_ - n), (0, 0))).reshape(-1)

    out_rows = _sc_combine(u_t, v_t, iu, iv, shift, np_, ke, out_c)

    return out_rows[:n].T[None, :, :, None]


# speedup vs baseline: 1308.3469x; 1.4085x over previous
import functools

import jax
import jax.numpy as jnp
from jax import lax
from jax.experimental import pallas as pl
from jax.experimental.pallas import tpu as pltpu
from jax.experimental.pallas import tpu_sc as plsc

_NC = 2
_NS = 16
_NW = _NC * _NS
_LANES = 16

_CHUNK = 8


def _mm_body(x_ref, a_ref, v_ref, u_out, v_out):
    xb = x_ref[...]
    u_out[...] = lax.dot_general(xb, a_ref[...], (((0,), (1,)), ((), ())),
                                 preferred_element_type=jnp.float32)
    v_out[...] = lax.dot_general(xb, v_ref[...], (((0,), (1,)), ((), ())),
                                 preferred_element_type=jnp.float32)


def _node_tables(xp, a, vw, np_, c, out_c, bn=512):
    grid = (np_ // bn,)
    return pl.pallas_call(
        _mm_body,
        grid=grid,
        in_specs=[
            pl.BlockSpec((c, bn), lambda i: (0, i)),
            pl.BlockSpec((out_c, c), lambda i: (0, 0)),
            pl.BlockSpec((out_c, c), lambda i: (0, 0)),
        ],
        out_specs=[
            pl.BlockSpec((bn, out_c), lambda i: (i, 0)),
            pl.BlockSpec((bn, out_c), lambda i: (i, 0)),
        ],
        out_shape=[
            jax.ShapeDtypeStruct((np_, out_c), jnp.float32),
            jax.ShapeDtypeStruct((np_, out_c), jnp.float32),
        ],
    )(xp, a, vw)


def _sc_combine(u_t, v_t, iu, iv, shift, np_, ke, out_c):
    nodes_per_w = np_ // _NW
    nchunks = nodes_per_w // _CHUNK
    nidx = _CHUNK * ke
    ngrp = out_c // _LANES
    assert nchunks % 2 == 0

    mesh = plsc.VectorSubcoreMesh(core_axis_name="c", subcore_axis_name="s")

    @functools.partial(
        pl.kernel,
        out_type=jax.ShapeDtypeStruct((np_, out_c), jnp.float32),
        mesh=mesh,
        scratch_types=[
            pltpu.VMEM((nchunks, nidx), jnp.int32),
            pltpu.VMEM((nchunks, nidx), jnp.int32),
            pltpu.VMEM((2, nidx, out_c), jnp.float32),
            pltpu.VMEM((2, nidx, out_c), jnp.float32),
            pltpu.VMEM((_CHUNK, out_c), jnp.float32),
            pltpu.VMEM((out_c,), jnp.float32),
            pltpu.SemaphoreType.DMA,
            pltpu.SemaphoreType.DMA,
            pltpu.SemaphoreType.DMA,
            pltpu.SemaphoreType.DMA,
        ],
    )
    def sc_kernel(u_hbm, v_hbm, iu_hbm, iv_hbm, shift_hbm, out_hbm,
                  iu_v, iv_v, urows, vrows, outrows, shift_v,
                  sem_u0, sem_u1, sem_v0, sem_v1):
        wid = lax.axis_index("s") * _NC + lax.axis_index("c")
        base = wid * nodes_per_w
        pltpu.sync_copy(shift_hbm, shift_v)
        pltpu.sync_copy(iu_hbm.at[wid], iu_v)
        pltpu.sync_copy(iv_hbm.at[wid], iv_v)
        sems = ((sem_u0, sem_v0), (sem_u1, sem_v1))

        def fire(f, buf):
            su, sv = sems[buf]
            pltpu.async_copy(u_hbm.at[iu_v.at[f]], urows.at[buf], su)
            pltpu.async_copy(v_hbm.at[iv_v.at[f]], vrows.at[buf], sv)

        def process(f, buf):
            su, sv = sems[buf]
            pltpu.make_async_copy(u_hbm.at[iu_v.at[f]], urows.at[buf], su).wait()
            pltpu.make_async_copy(v_hbm.at[iv_v.at[f]], vrows.at[buf], sv).wait()
            ub = urows.at[buf]
            vb = vrows.at[buf]

            def node_body(n, carry2):
                e0 = n * ke
                for g in range(ngrp):
                    sl = pl.ds(g * _LANES, _LANES)
                    m = ub[e0, sl] + vb[e0, sl]
                    for e in range(1, ke):
                        m = jnp.maximum(m, ub[e0 + e, sl] + vb[e0 + e, sl])
                    outrows[n, sl] = jnp.maximum(m + shift_v[sl], 0.0)
                return carry2

            lax.fori_loop(0, _CHUNK, node_body, 0, unroll=False)
            pltpu.sync_copy(outrows, out_hbm.at[pl.ds(base + f * _CHUNK, _CHUNK)])

        fire(0, 0)

        def pair_body(i, carry):
            ci2 = 2 * i
            fire(ci2 + 1, 1)
            process(ci2, 0)

            @pl.when(ci2 + 2 < nchunks)
            def _():
                fire(ci2 + 2, 0)

            process(ci2 + 1, 1)
            return carry

        lax.fori_loop(0, nchunks // 2, pair_body, 0, unroll=False)

    return sc_kernel(u_t, v_t, iu, iv, shift)


def kernel(x, x_0, edge_index, W, b, gamma, beta, running_mean, running_var):
    _, c, n, _ = x.shape
    out_c = W.shape[0]
    k = edge_index.shape[-1]
    ke = k - 1

    scale = gamma / jnp.sqrt(running_var + 1e-5)
    shift = (b - running_mean) * scale + beta
    w1 = W[:, :c]
    w2 = W[:, c:]
    a = (w1 - w2) * scale[:, None]
    vw = w2 * scale[:, None]

    align = _NW * _CHUNK
    np_ = ((n + align - 1) // align) * align

    xt = x[0, :, :, 0]
    xp = jnp.pad(xt, ((0, 0), (0, np_ - n)))

    u_t, v_t = _node_tables(xp, a, vw, np_, c, out_c)

    nodes_per_w = np_ // _NW
    nchunks = nodes_per_w // _CHUNK
    ei = edge_index.astype(jnp.int32)
    iu = jnp.pad(ei[1, 0, :, 1:], ((0, np_ - n), (0, 0))).reshape(
        _NW, nchunks, _CHUNK * ke)
    iv = jnp.pad(ei[0, 0, :, 1:], ((0, np_ - n), (0, 0))).reshape(
        _NW, nchunks, _CHUNK * ke)

    out_rows = _sc_combine(u_t, v_t, iu, iv, shift, np_, ke, out_c)

    return out_rows[:n].T[None, :, :, None]


# speedup vs baseline: 1765.9475x; 1.3498x over previous
import functools

import jax
import jax.numpy as jnp
from jax import lax
from jax.experimental import pallas as pl
from jax.experimental.pallas import tpu as pltpu
from jax.experimental.pallas import tpu_sc as plsc

_NC = 2
_NS = 16
_NW = _NC * _NS
_BLANES = 32

_CHUNK = 8


def _mm_body(x_ref, a_ref, v_ref, u_out, v_out):
    xb = x_ref[...]
    u_out[...] = lax.dot_general(xb, a_ref[...], (((0,), (1,)), ((), ())),
                                 preferred_element_type=jnp.float32
                                 ).astype(jnp.bfloat16)
    v_out[...] = lax.dot_general(xb, v_ref[...], (((0,), (1,)), ((), ())),
                                 preferred_element_type=jnp.float32
                                 ).astype(jnp.bfloat16)


def _node_tables(xp, a, vw, np_, c, out_c, bn=512):
    grid = (np_ // bn,)
    return pl.pallas_call(
        _mm_body,
        grid=grid,
        in_specs=[
            pl.BlockSpec((c, bn), lambda i: (0, i)),
            pl.BlockSpec((out_c, c), lambda i: (0, 0)),
            pl.BlockSpec((out_c, c), lambda i: (0, 0)),
        ],
        out_specs=[
            pl.BlockSpec((bn, out_c), lambda i: (i, 0)),
            pl.BlockSpec((bn, out_c), lambda i: (i, 0)),
        ],
        out_shape=[
            jax.ShapeDtypeStruct((np_, out_c), jnp.bfloat16),
            jax.ShapeDtypeStruct((np_, out_c), jnp.bfloat16),
        ],
    )(xp, a, vw)


def _sc_combine(u_t, v_t, iu, iv, shift_bf, np_, ke, out_c):
    nodes_per_w = np_ // _NW
    nchunks = nodes_per_w // _CHUNK
    nidx = _CHUNK * ke
    ngrp = out_c // _BLANES
    assert nchunks % 2 == 0 and nidx <= 128

    mesh = plsc.VectorSubcoreMesh(core_axis_name="c", subcore_axis_name="s")

    @functools.partial(
        pl.kernel,
        out_type=jax.ShapeDtypeStruct((np_, out_c), jnp.bfloat16),
        mesh=mesh,
        compiler_params=pltpu.CompilerParams(use_tc_tiling_on_sc=False),
        scratch_types=[
            pltpu.VMEM((nchunks, nidx), jnp.int32),
            pltpu.VMEM((nchunks, nidx), jnp.int32),
            pltpu.VMEM((2, nidx, out_c), jnp.bfloat16),
            pltpu.VMEM((2, nidx, out_c), jnp.bfloat16),
            pltpu.VMEM((_CHUNK, out_c), jnp.bfloat16),
            pltpu.VMEM((out_c,), jnp.bfloat16),
            pltpu.SemaphoreType.DMA,
            pltpu.SemaphoreType.DMA,
            pltpu.SemaphoreType.DMA,
            pltpu.SemaphoreType.DMA,
        ],
    )
    def sc_kernel(u_hbm, v_hbm, iu_hbm, iv_hbm, sh_hbm, out_hbm,
                  iu_v, iv_v, urows, vrows, outrows, sh_v,
                  sem_u0, sem_u1, sem_v0, sem_v1):
        wid = lax.axis_index("s") * _NC + lax.axis_index("c")
        base = wid * nodes_per_w
        pltpu.sync_copy(sh_hbm, sh_v)
        pltpu.sync_copy(iu_hbm.at[wid], iu_v)
        pltpu.sync_copy(iv_hbm.at[wid], iv_v)
        sems = ((sem_u0, sem_v0), (sem_u1, sem_v1))

        def fire(f, buf):
            su, sv = sems[buf]
            pltpu.async_copy(u_hbm.at[iu_v.at[f]], urows.at[buf], su)
            pltpu.async_copy(v_hbm.at[iv_v.at[f]], vrows.at[buf], sv)

        def process(f, buf):
            su, sv = sems[buf]
            pltpu.make_async_copy(u_hbm.at[iu_v.at[f]], urows.at[buf], su).wait()
            pltpu.make_async_copy(v_hbm.at[iv_v.at[f]], vrows.at[buf], sv).wait()
            ub = urows.at[buf]
            vb = vrows.at[buf]
            zero = jnp.zeros((_BLANES,), jnp.bfloat16)

            def node_body(n, carry2):
                e0 = n * ke

                for g in range(ngrp):
                    sl = pl.ds(g * _BLANES, _BLANES)
                    m = ub[e0, sl] + vb[e0, sl]
                    for e in range(1, ke):
                        m = jnp.maximum(m, ub[e0 + e, sl] + vb[e0 + e, sl])
                    outrows[n, sl] = jnp.maximum(m + sh_v[sl], zero)
                return carry2

            lax.fori_loop(0, _CHUNK, node_body, 0, unroll=False)
            pltpu.sync_copy(outrows, out_hbm.at[pl.ds(base + f * _CHUNK, _CHUNK)])

        fire(0, 0)

        def pair_body(i, carry):
            ci2 = 2 * i
            fire(ci2 + 1, 1)
            process(ci2, 0)

            @pl.when(ci2 + 2 < nchunks)
            def _():
                fire(ci2 + 2, 0)

            process(ci2 + 1, 1)
            return carry

        lax.fori_loop(0, nchunks // 2, pair_body, 0, unroll=False)

    return sc_kernel(u_t, v_t, iu, iv, shift_bf)


def kernel(x, x_0, edge_index, W, b, gamma, beta, running_mean, running_var):
    _, c, n, _ = x.shape
    out_c = W.shape[0]
    k = edge_index.shape[-1]

    scale = gamma / jnp.sqrt(running_var + 1e-5)
    shift = (b - running_mean) * scale + beta
    w1 = W[:, :c]
    w2 = W[:, c:]
    a = (w1 - w2) * scale[:, None]
    vw = w2 * scale[:, None]

    align = _NW * _CHUNK
    np_ = ((n + align - 1) // align) * align

    xt = x[0, :, :, 0]
    xp = jnp.pad(xt, ((0, 0), (0, np_ - n)))

    u_t, v_t = _node_tables(xp, a, vw, np_, c, out_c)

    nodes_per_w = np_ // _NW
    nchunks = nodes_per_w // _CHUNK
    ke = k - 1
    ei = edge_index.astype(jnp.int32)[:, :, :, 1:]
    iu = jnp.pad(ei[1, 0], ((0, np_ - n), (0, 0))).reshape(
        _NW, nchunks, _CHUNK * ke)
    iv = jnp.pad(ei[0, 0], ((0, np_ - n), (0, 0))).reshape(
        _NW, nchunks, _CHUNK * ke)

    out_rows = _sc_combine(u_t, v_t, iu, iv, shift.astype(jnp.bfloat16),
                           np_, ke, out_c)
    return out_rows[:n].astype(jnp.float32).T[None, :, :, None]


# speedup vs baseline: 1850.9637x; 1.0481x over previous
import functools

import jax
import jax.numpy as jnp
from jax import lax
from jax.experimental import pallas as pl
from jax.experimental.pallas import tpu as pltpu
from jax.experimental.pallas import tpu_sc as plsc

_NC = 2
_NS = 16
_NW = _NC * _NS
_BLANES = 32

_CHUNK = 8


def _mm_body(x_ref, a_ref, v_ref, u_out, v_out):
    xb = x_ref[...]
    u_out[...] = lax.dot_general(xb, a_ref[...], (((0,), (1,)), ((), ())),
                                 preferred_element_type=jnp.float32
                                 ).astype(jnp.bfloat16)
    v_out[...] = lax.dot_general(xb, v_ref[...], (((0,), (1,)), ((), ())),
                                 preferred_element_type=jnp.float32
                                 ).astype(jnp.bfloat16)


def _node_tables(xp, a, vw, np_, c, out_c, bn=None):
    if bn is None:
        bn = np_
    grid = (np_ // bn,)
    return pl.pallas_call(
        _mm_body,
        grid=grid,
        in_specs=[
            pl.BlockSpec((c, bn), lambda i: (0, i)),
            pl.BlockSpec((out_c, c), lambda i: (0, 0)),
            pl.BlockSpec((out_c, c), lambda i: (0, 0)),
        ],
        out_specs=[
            pl.BlockSpec((bn, out_c), lambda i: (i, 0)),
            pl.BlockSpec((bn, out_c), lambda i: (i, 0)),
        ],
        out_shape=[
            jax.ShapeDtypeStruct((np_, out_c), jnp.bfloat16),
            jax.ShapeDtypeStruct((np_, out_c), jnp.bfloat16),
        ],
    )(xp, a, vw)


def _sc_combine(u_t, v_t, iu, iv, shift_bf, np_, k, out_c):
    nodes_per_w = np_ // _NW
    nchunks = nodes_per_w // _CHUNK
    nidx = _CHUNK * k
    ngrp = out_c // _BLANES
    assert nchunks % 2 == 0 and nidx <= 128

    mesh = plsc.VectorSubcoreMesh(core_axis_name="c", subcore_axis_name="s")

    @functools.partial(
        pl.kernel,
        out_type=jax.ShapeDtypeStruct((np_, out_c), jnp.bfloat16),
        mesh=mesh,
        compiler_params=pltpu.CompilerParams(use_tc_tiling_on_sc=False),
        scratch_types=[
            pltpu.VMEM((nchunks, nidx), jnp.int32),
            pltpu.VMEM((nchunks, nidx), jnp.int32),
            pltpu.VMEM((2, nidx, out_c), jnp.bfloat16),
            pltpu.VMEM((2, nidx, out_c), jnp.bfloat16),
            pltpu.VMEM((_CHUNK, out_c), jnp.bfloat16),
            pltpu.VMEM((out_c,), jnp.bfloat16),
            pltpu.SemaphoreType.DMA,
            pltpu.SemaphoreType.DMA,
            pltpu.SemaphoreType.DMA,
            pltpu.SemaphoreType.DMA,
        ],
    )
    def sc_kernel(u_hbm, v_hbm, iu_hbm, iv_hbm, sh_hbm, out_hbm,
                  iu_v, iv_v, urows, vrows, outrows, sh_v,
                  sem_u0, sem_u1, sem_v0, sem_v1):
        wid = lax.axis_index("s") * _NC + lax.axis_index("c")
        base = wid * nodes_per_w
        pltpu.sync_copy(sh_hbm, sh_v)
        pltpu.sync_copy(iu_hbm.at[wid], iu_v)
        pltpu.sync_copy(iv_hbm.at[wid], iv_v)
        sems = ((sem_u0, sem_v0), (sem_u1, sem_v1))

        def fire(f, buf):
            su, sv = sems[buf]
            pltpu.async_copy(u_hbm.at[iu_v.at[f]], urows.at[buf], su)
            pltpu.async_copy(v_hbm.at[iv_v.at[f]], vrows.at[buf], sv)

        def process(f, buf):
            su, sv = sems[buf]
            pltpu.make_async_copy(u_hbm.at[iu_v.at[f]], urows.at[buf], su).wait()
            pltpu.make_async_copy(v_hbm.at[iv_v.at[f]], vrows.at[buf], sv).wait()
            ub = urows.at[buf]
            vb = vrows.at[buf]
            zero = jnp.zeros((_BLANES,), jnp.bfloat16)

            def node_body(n, carry2):
                e0 = n * k

                for g in range(ngrp):
                    sl = pl.ds(g * _BLANES, _BLANES)
                    m = ub[e0 + 1, sl] + vb[e0 + 1, sl]
                    for e in range(2, k):
                        m = jnp.maximum(m, ub[e0 + e, sl] + vb[e0 + e, sl])
                    outrows[n, sl] = jnp.maximum(m + sh_v[sl], zero)
                return carry2

            lax.fori_loop(0, _CHUNK, node_body, 0, unroll=False)
            pltpu.sync_copy(outrows, out_hbm.at[pl.ds(base + f * _CHUNK, _CHUNK)])

        fire(0, 0)

        def pair_body(i, carry):
            ci2 = 2 * i
            fire(ci2 + 1, 1)
            process(ci2, 0)

            @pl.when(ci2 + 2 < nchunks)
            def _():
                fire(ci2 + 2, 0)

            process(ci2 + 1, 1)
            return carry

        lax.fori_loop(0, nchunks // 2, pair_body, 0, unroll=False)

    return sc_kernel(u_t, v_t, iu, iv, shift_bf)


def kernel(x, x_0, edge_index, W, b, gamma, beta, running_mean, running_var):
    _, c, n, _ = x.shape
    out_c = W.shape[0]
    k = edge_index.shape[-1]

    scale = gamma / jnp.sqrt(running_var + 1e-5)
    shift = (b - running_mean) * scale + beta
    w1 = W[:, :c]
    w2 = W[:, c:]
    a = (w1 - w2) * scale[:, None]
    vw = w2 * scale[:, None]

    align = _NW * _CHUNK
    np_ = ((n + align - 1) // align) * align

    xt = x[0, :, :, 0]
    u_t, v_t = _node_tables(xt, a, vw, n, c, out_c)

    nodes_per_w = np_ // _NW
    nchunks = nodes_per_w // _CHUNK
    ei = edge_index.astype(jnp.int32)
    iu = jnp.pad(ei[1, 0], ((0, np_ - n), (0, 0))).reshape(
        _NW, nchunks, _CHUNK * k)
    iv = jnp.pad(ei[0, 0], ((0, np_ - n), (0, 0))).reshape(
        _NW, nchunks, _CHUNK * k)

    out_rows = _sc_combine(u_t, v_t, iu, iv, shift.astype(jnp.bfloat16),
                           np_, k, out_c)
    return out_rows[:n].astype(jnp.float32).T[None, :, :, None]


# speedup vs baseline: 1956.6907x; 1.0571x over previous
import functools

import jax
import jax.numpy as jnp
from jax import lax
from jax.experimental import pallas as pl
from jax.experimental.pallas import tpu as pltpu
from jax.experimental.pallas import tpu_sc as plsc

_NC = 2
_NS = 16
_NW = _NC * _NS
_BLANES = 32

_CHUNK = 8

_N0 = 54
_N1 = 26


def _mm_body(x_ref, a_ref, v_ref, u_out, v_out):
    xb = x_ref[...]
    u_out[...] = lax.dot_general(xb, a_ref[...], (((0,), (1,)), ((), ())),
                                 preferred_element_type=jnp.float32
                                 ).astype(jnp.bfloat16)
    v_out[...] = lax.dot_general(xb, v_ref[...], (((0,), (1,)), ((), ())),
                                 preferred_element_type=jnp.float32
                                 ).astype(jnp.bfloat16)


def _node_tables(xp, a, vw, np_, c, out_c, bn=None):
    if bn is None:
        bn = np_
    grid = (np_ // bn,)
    return pl.pallas_call(
        _mm_body,
        grid=grid,
        in_specs=[
            pl.BlockSpec((c, bn), lambda i: (0, i)),
            pl.BlockSpec((out_c, c), lambda i: (0, 0)),
            pl.BlockSpec((out_c, c), lambda i: (0, 0)),
        ],
        out_specs=[
            pl.BlockSpec((bn, out_c), lambda i: (i, 0)),
            pl.BlockSpec((bn, out_c), lambda i: (i, 0)),
        ],
        out_shape=[
            jax.ShapeDtypeStruct((np_, out_c), jnp.bfloat16),
            jax.ShapeDtypeStruct((np_, out_c), jnp.bfloat16),
        ],
    )(xp, a, vw)


def _sc_combine(u_t, v_t, iu, iv, shift_bf, np_, k, out_c):
    nidx = _CHUNK * k
    ngrp = out_c // _BLANES
    nmax = max(_N0, _N1)
    npair = _N0 + _N1
    assert _N0 % 2 == 0 and _N1 % 2 == 0 and nidx <= 128
    assert np_ == _NS * npair * _CHUNK

    mesh = plsc.VectorSubcoreMesh(core_axis_name="c", subcore_axis_name="s")

    @functools.partial(
        pl.kernel,
        out_type=jax.ShapeDtypeStruct((np_, out_c), jnp.bfloat16),
        mesh=mesh,
        compiler_params=pltpu.CompilerParams(use_tc_tiling_on_sc=False),
        scratch_types=[
            pltpu.VMEM((nmax, nidx), jnp.int32),
            pltpu.VMEM((nmax, nidx), jnp.int32),
            pltpu.VMEM((2, nidx, out_c), jnp.bfloat16),
            pltpu.VMEM((2, nidx, out_c), jnp.bfloat16),
            pltpu.VMEM((_CHUNK, out_c), jnp.bfloat16),
            pltpu.VMEM((out_c,), jnp.bfloat16),
            pltpu.SemaphoreType.DMA,
            pltpu.SemaphoreType.DMA,
            pltpu.SemaphoreType.DMA,
            pltpu.SemaphoreType.DMA,
        ],
    )
    def sc_kernel(u_hbm, v_hbm, iu_hbm, iv_hbm, sh_hbm, out_hbm,
                  iu_v, iv_v, urows, vrows, outrows, sh_v,
                  sem_u0, sem_u1, sem_v0, sem_v1):
        s = lax.axis_index("s")
        cidx = lax.axis_index("c")
        nchunks = jnp.where(cidx == 0, _N0, _N1)
        cbase = s * npair + cidx * _N0
        base = cbase * _CHUNK
        pltpu.sync_copy(sh_hbm, sh_v)
        pltpu.sync_copy(iu_hbm.at[pl.ds(cbase, nmax)], iu_v)
        pltpu.sync_copy(iv_hbm.at[pl.ds(cbase, nmax)], iv_v)
        sems = ((sem_u0, sem_v0), (sem_u1, sem_v1))

        def fire(f, buf):
            su, sv = sems[buf]
            pltpu.async_copy(u_hbm.at[iu_v.at[f]], urows.at[buf], su)
            pltpu.async_copy(v_hbm.at[iv_v.at[f]], vrows.at[buf], sv)

        def process(f, buf):
            su, sv = sems[buf]
            pltpu.make_async_copy(u_hbm.at[iu_v.at[f]], urows.at[buf], su).wait()
            pltpu.make_async_copy(v_hbm.at[iv_v.at[f]], vrows.at[buf], sv).wait()
            ub = urows.at[buf]
            vb = vrows.at[buf]
            zero = jnp.zeros((_BLANES,), jnp.bfloat16)

            def node_body(n, carry2):
                e0 = n * k

                for g in range(ngrp):
                    sl = pl.ds(g * _BLANES, _BLANES)
                    m = ub[e0 + 1, sl] + vb[e0 + 1, sl]
                    for e in range(2, k):
                        m = jnp.maximum(m, ub[e0 + e, sl] + vb[e0 + e, sl])
                    outrows[n, sl] = jnp.maximum(m + sh_v[sl], zero)
                return carry2

            lax.fori_loop(0, _CHUNK, node_body, 0, unroll=False)
            pltpu.sync_copy(outrows, out_hbm.at[pl.ds(base + f * _CHUNK, _CHUNK)])

        fire(0, 0)

        def pair_body(i, carry):
            ci2 = 2 * i
            fire(ci2 + 1, 1)
            process(ci2, 0)

            @pl.when(ci2 + 2 < nchunks)
            def _():
                fire(ci2 + 2, 0)

            process(ci2 + 1, 1)
            return carry

        lax.fori_loop(0, nchunks // 2, pair_body, 0, unroll=False)

    return sc_kernel(u_t, v_t, iu, iv, shift_bf)


def kernel(x, x_0, edge_index, W, b, gamma, beta, running_mean, running_var):
    _, c, n, _ = x.shape
    out_c = W.shape[0]
    k = edge_index.shape[-1]

    scale = gamma / jnp.sqrt(running_var + 1e-5)
    shift = (b - running_mean) * scale + beta
    w1 = W[:, :c]
    w2 = W[:, c:]
    a = (w1 - w2) * scale[:, None]
    vw = w2 * scale[:, None]

    align = _NW * _CHUNK
    np_ = ((n + align - 1) // align) * align

    xt = x[0, :, :, 0]
    u_t, v_t = _node_tables(xt, a, vw, n, c, out_c)

    nmax = max(_N0, _N1)
    rows_flat = max(np_ // _CHUNK, (_NS - 1) * (_N0 + _N1) + _N0 + nmax)
    ei = edge_index.astype(jnp.int32)
    iu = jnp.pad(ei[1, 0], ((0, rows_flat * _CHUNK - n), (0, 0))).reshape(
        rows_flat, _CHUNK * k)
    iv = jnp.pad(ei[0, 0], ((0, rows_flat * _CHUNK - n), (0, 0))).reshape(
        rows_flat, _CHUNK * k)

    out_rows = _sc_combine(u_t, v_t, iu, iv, shift.astype(jnp.bfloat16),
                           np_, k, out_c)
    return out_rows[:n].astype(jnp.float32).T[None, :, :, None]
